# Initial kernel scaffold; baseline (speedup 1.0000x reference)
import jax
import jax.numpy as jnp
from jax.experimental import pallas as pl


def kernel(edge_attr, idx_sender, idx_receiver, x_sender, eW1, eb1, eW2, eg, eb, rW1, rb1, rW2, rg, rb):
    raise NotImplementedError("write your pallas kernel here")


# speedup vs baseline: 2.7012x; 2.7012x over previous
import dataclasses
import functools

import jax
import jax.numpy as jnp
from jax import lax
from jax.experimental import pallas as pl
from jax.experimental.pallas import tpu as pltpu
from jax.experimental.pallas import tpu_sc as plsc

N_HID = 128
N_NODES = 10000
N_EDGES = 320000

NC = 2
NS = 16
NW = NC * NS
EPW = N_EDGES // NW
CB = 80
NPAD = 10240
NPT = NPAD // NS

_MESH = plsc.VectorSubcoreMesh(core_axis_name="c", subcore_axis_name="s")
_CP = pltpu.CompilerParams()
if "needs_layout_passes" in pltpu.CompilerParams.__dataclass_fields__:
    _CP = dataclasses.replace(_CP, needs_layout_passes=False)


def _proj_body(x_ref, wb_ref, wc_ref, xb_ref, xc_ref):
    x = x_ref[...]
    xb_ref[...] = jnp.dot(x, wb_ref[...], preferred_element_type=jnp.float32)
    xc_ref[...] = jnp.dot(x, wc_ref[...], preferred_element_type=jnp.float32)


def _project(x, wb, wc):
    blk = 2000
    return pl.pallas_call(
        _proj_body,
        grid=(N_NODES // blk,),
        in_specs=[
            pl.BlockSpec((blk, N_HID), lambda i: (i, 0)),
            pl.BlockSpec((N_HID, N_HID), lambda i: (0, 0)),
            pl.BlockSpec((N_HID, N_HID), lambda i: (0, 0)),
        ],
        out_specs=[
            pl.BlockSpec((blk, N_HID), lambda i: (i, 0)),
            pl.BlockSpec((blk, N_HID), lambda i: (i, 0)),
        ],
        out_shape=[jax.ShapeDtypeStruct((N_NODES, N_HID), jnp.float32)] * 2,
    )(x, wb, wc)


@functools.partial(
    pl.kernel,
    out_type=[
        jax.ShapeDtypeStruct((N_EDGES, N_HID), jnp.float32),
        jax.ShapeDtypeStruct((N_EDGES, N_HID), jnp.float32),
        jax.ShapeDtypeStruct((NW * NPAD,), jnp.float32),
    ],
    mesh=_MESH,
    compiler_params=_CP,
    scratch_types=[
        pltpu.VMEM((CB,), jnp.int32),
        pltpu.VMEM((CB,), jnp.int32),
        pltpu.VMEM((CB, N_HID), jnp.float32),
        pltpu.VMEM((NPAD,), jnp.float32),
    ],
)
def _gather_kernel(xb_hbm, xc_hbm, idxs_hbm, idxr_hbm,
                   xbg_hbm, xcg_hbm, cnt_hbm,
                   idxs_v, idxr_v, buf_v, cnt_v):
    c = lax.axis_index("c")
    s = lax.axis_index("s")
    w = c * NS + s
    zero16 = jnp.zeros((16,), jnp.float32)
    one16 = zero16 + 1.0

    @pl.loop(0, NPAD, step=16)
    def _(i):
        cnt_v[pl.ds(i, 16)] = zero16

    base = w * EPW

    @pl.loop(0, EPW, step=CB)
    def _(k):
        o = base + k
        pltpu.sync_copy(idxs_hbm.at[pl.ds(o, CB)], idxs_v)
        pltpu.sync_copy(idxr_hbm.at[pl.ds(o, CB)], idxr_v)
        pltpu.sync_copy(xb_hbm.at[idxs_v], buf_v)
        pltpu.sync_copy(buf_v, xbg_hbm.at[pl.ds(o, CB)])
        pltpu.sync_copy(xc_hbm.at[idxr_v], buf_v)
        pltpu.sync_copy(buf_v, xcg_hbm.at[pl.ds(o, CB)])

        @pl.loop(0, CB, step=16)
        def _(j):
            idx16 = idxr_v[pl.ds(j, 16)]
            plsc.addupdate_scatter(cnt_v, [idx16], one16)

    pltpu.sync_copy(cnt_v, cnt_hbm.at[pl.ds(w * NPAD, NPAD)])


def _edge_body(ea_ref, gb_ref, gc_ref, w1_ref, w2_ref, b1_ref, g_ref, b_ref,
               nea_ref, eu_ref):
    ea = ea_ref[...]
    h = jnp.dot(ea, w1_ref[...], preferred_element_type=jnp.float32)
    h = h + gb_ref[...] + gc_ref[...] + b1_ref[...]
    h = h * jax.nn.sigmoid(h)
    h = jnp.dot(h, w2_ref[...], preferred_element_type=jnp.float32)
    mu = jnp.mean(h, axis=-1, keepdims=True)
    d = h - mu
    var = jnp.mean(d * d, axis=-1, keepdims=True)
    eu = (d * lax.rsqrt(var + 1e-5)) * g_ref[...] + b_ref[...]
    eu_ref[...] = eu
    nea_ref[...] = ea + eu


def _edge(ea, gb, gc, w1a, w2, b1, g, b):
    blk = 2000
    row = lambda i: (i, 0)
    full = lambda i: (0, 0)
    return pl.pallas_call(
        _edge_body,
        grid=(N_EDGES // blk,),
        in_specs=[
            pl.BlockSpec((blk, N_HID), row),
            pl.BlockSpec((blk, N_HID), row),
            pl.BlockSpec((blk, N_HID), row),
            pl.BlockSpec((N_HID, N_HID), full),
            pl.BlockSpec((N_HID, N_HID), full),
            pl.BlockSpec((1, N_HID), full),
            pl.BlockSpec((1, N_HID), full),
            pl.BlockSpec((1, N_HID), full),
        ],
        out_specs=[
            pl.BlockSpec((blk, N_HID), row),
            pl.BlockSpec((blk, N_HID), row),
        ],
        out_shape=[jax.ShapeDtypeStruct((N_EDGES, N_HID), jnp.float32)] * 2,
    )(ea, gb, gc, w1a, w2, b1, g, b)


@functools.partial(
    pl.kernel,
    out_type=jax.ShapeDtypeStruct((NC, NPAD, N_HID), jnp.float32),
    mesh=_MESH,
    scratch_types=[
        pltpu.VMEM((CB, N_HID), jnp.float32),
        pltpu.VMEM((CB,), jnp.int32),
        pltpu.VMEM((80, N_HID), jnp.float32),
        pltpu.VMEM_SHARED((NPAD, N_HID), jnp.float32),
    ],
)
def _scatter_kernel(eu_hbm, idxr_hbm, sums_hbm, eub_v, idx_v, dump_v, sums_sh):
    c = lax.axis_index("c")
    s = lax.axis_index("s")
    w = c * NS + s
    zero16 = jnp.zeros((16,), jnp.float32)

    @pl.loop(0, CB)
    def _(i):
        @pl.loop(0, N_HID, step=16)
        def _(j):
            eub_v[i, pl.ds(j, 16)] = zero16

    @pl.loop(0, NPT, step=CB)
    def _(r):
        pltpu.sync_copy(eub_v, sums_sh.at[pl.ds(s * NPT + r, CB)])
    plsc.subcore_barrier()

    base = w * EPW

    @pl.loop(0, EPW, step=CB)
    def _(k):
        o = base + k
        pltpu.sync_copy(idxr_hbm.at[pl.ds(o, CB)], idx_v)
        pltpu.sync_copy(eu_hbm.at[pl.ds(o, CB)], eub_v)
        pltpu.sync_copy(eub_v, sums_sh.at[idx_v], add=True)

    plsc.subcore_barrier()

    @pl.loop(0, NPT, step=80)
    def _(r):
        pltpu.sync_copy(sums_sh.at[pl.ds(s * NPT + r, 80)], dump_v)
        pltpu.sync_copy(dump_v, sums_hbm.at[c, pl.ds(s * NPT + r, 80)])


def _node_body(x_ref, sums_ref, cnt_ref, w1x_ref, w1c_ref, b1_ref, w2_ref,
               g_ref, b_ref, nx_ref):
    x = x_ref[...]
    ssum = sums_ref[0] + sums_ref[1]
    cnt = jnp.sum(cnt_ref[...], axis=1)[:, None]
    coll = ssum / jnp.maximum(cnt, 1.0)
    h = jnp.dot(x, w1x_ref[...], preferred_element_type=jnp.float32)
    h = h + jnp.dot(coll, w1c_ref[...], preferred_element_type=jnp.float32)
    h = h + b1_ref[...]
    h = h * jax.nn.sigmoid(h)
    h = jnp.dot(h, w2_ref[...], preferred_element_type=jnp.float32)
    mu = jnp.mean(h, axis=-1, keepdims=True)
    d = h - mu
    var = jnp.mean(d * d, axis=-1, keepdims=True)
    nx_ref[...] = x + (d * lax.rsqrt(var + 1e-5)) * g_ref[...] + b_ref[...]


def _node(x, sums, cnt, w1x, w1c, b1, w2, g, b):
    blk = 2000
    row = lambda i: (i, 0)
    full = lambda i: (0, 0)
    return pl.pallas_call(
        _node_body,
        grid=(N_NODES // blk,),
        in_specs=[
            pl.BlockSpec((blk, N_HID), row),
            pl.BlockSpec((NC, blk, N_HID), lambda i: (0, i, 0)),
            pl.BlockSpec((blk, NW), lambda i: (i, 0)),
            pl.BlockSpec((N_HID, N_HID), full),
            pl.BlockSpec((N_HID, N_HID), full),
            pl.BlockSpec((1, N_HID), full),
            pl.BlockSpec((N_HID, N_HID), full),
            pl.BlockSpec((1, N_HID), full),
            pl.BlockSpec((1, N_HID), full),
        ],
        out_specs=pl.BlockSpec((blk, N_HID), row),
        out_shape=jax.ShapeDtypeStruct((N_NODES, N_HID), jnp.float32),
    )(x, sums, cnt, w1x, w1c, b1, w2, g, b)


def kernel(edge_attr, idx_sender, idx_receiver, x_sender,
           eW1, eb1, eW2, eg, eb, rW1, rb1, rW2, rg, rb):
    ea = edge_attr.reshape(N_EDGES, N_HID)
    x = x_sender.reshape(N_NODES, N_HID)
    idx_s = idx_sender.astype(jnp.int32)
    idx_r = idx_receiver.astype(jnp.int32)

    w1a = eW1[:N_HID]
    xb, xc = _project(x, eW1[N_HID:2 * N_HID], eW1[2 * N_HID:])
    xbg, xcg, cnt = _gather_kernel(xb, xc, idx_s, idx_r)
    nea, eu = _edge(ea, xbg, xcg, w1a, eW2,
                    eb1.reshape(1, -1), eg.reshape(1, -1), eb.reshape(1, -1))
    sums = _scatter_kernel(eu, idx_r)
    nx = _node(x, sums, cnt.reshape(NW, NPAD).T, rW1[:N_HID], rW1[N_HID:],
               rb1.reshape(1, -1), rW2, rg.reshape(1, -1), rb.reshape(1, -1))
    return (nx.reshape(1, N_NODES, N_HID), nea.reshape(1, N_EDGES, N_HID))


# speedup vs baseline: 3.4249x; 1.2679x over previous
import dataclasses
import functools

import jax
import jax.numpy as jnp
from jax import lax
from jax.experimental import pallas as pl
from jax.experimental.pallas import tpu as pltpu
from jax.experimental.pallas import tpu_sc as plsc

N_HID = 128
N_NODES = 10000
N_EDGES = 320000

NC = 2
NS = 16
NW = NC * NS
EPW = N_EDGES // NW
CB = 40
PB = 2 * CB
NPAIR = EPW // PB
NPAD = 10240
NPT = NPAD // NS

_MESH = plsc.VectorSubcoreMesh(core_axis_name="c", subcore_axis_name="s")
_CP = pltpu.CompilerParams()
if "needs_layout_passes" in pltpu.CompilerParams.__dataclass_fields__:
    _CP = dataclasses.replace(_CP, needs_layout_passes=False)


def _proj_body(x_ref, wb_ref, wc_ref, xb_ref, xc_ref):
    x = x_ref[...]
    xb_ref[...] = jnp.dot(x, wb_ref[...], preferred_element_type=jnp.float32)
    xc_ref[...] = jnp.dot(x, wc_ref[...], preferred_element_type=jnp.float32)


def _project(x, wb, wc):
    blk = 2000
    return pl.pallas_call(
        _proj_body,
        grid=(N_NODES // blk,),
        in_specs=[
            pl.BlockSpec((blk, N_HID), lambda i: (i, 0)),
            pl.BlockSpec((N_HID, N_HID), lambda i: (0, 0)),
            pl.BlockSpec((N_HID, N_HID), lambda i: (0, 0)),
        ],
        out_specs=[
            pl.BlockSpec((blk, N_HID), lambda i: (i, 0)),
            pl.BlockSpec((blk, N_HID), lambda i: (i, 0)),
        ],
        out_shape=[jax.ShapeDtypeStruct((N_NODES, N_HID), jnp.float32)] * 2,
    )(x, wb, wc)


@functools.partial(
    pl.kernel,
    out_type=[
        jax.ShapeDtypeStruct((N_EDGES, N_HID), jnp.float32),
        jax.ShapeDtypeStruct((N_EDGES, N_HID), jnp.float32),
        jax.ShapeDtypeStruct((NW * NPAD,), jnp.float32),
    ],
    mesh=_MESH,
    compiler_params=_CP,
    scratch_types=[
        pltpu.VMEM((1, PB), jnp.int32),
        pltpu.VMEM((1, PB), jnp.int32),
        pltpu.VMEM((CB, N_HID), jnp.float32),
        pltpu.VMEM((CB, N_HID), jnp.float32),
        pltpu.VMEM((CB, N_HID), jnp.float32),
        pltpu.VMEM((CB, N_HID), jnp.float32),
        pltpu.VMEM((NPAD,), jnp.float32),
        pltpu.SemaphoreType.DMA,
        pltpu.SemaphoreType.DMA,
        pltpu.SemaphoreType.DMA,
        pltpu.SemaphoreType.DMA,
        pltpu.SemaphoreType.DMA,
        pltpu.SemaphoreType.DMA,
        pltpu.SemaphoreType.DMA,
        pltpu.SemaphoreType.DMA,
        pltpu.SemaphoreType.DMA,
        pltpu.SemaphoreType.DMA,
    ],
)
def _gather_kernel(xb_hbm, xc_hbm, idxs_hbm, idxr_hbm,
                   xbg_hbm, xcg_hbm, cnt_hbm,
                   is_v, ir_v, ba0, ba1, bb0, bb1, cnt_v,
                   sis, sir, ga0, ga1, gb0, gb1, wa0, wa1, wb0, wb1):
    c = lax.axis_index("c")
    s = lax.axis_index("s")
    w = c * NS + s
    zero16 = jnp.zeros((16,), jnp.float32)
    one16 = zero16 + 1.0

    @pl.loop(0, NPAD, step=16)
    def _(i):
        cnt_v[pl.ds(i, 16)] = zero16

    base = w * EPW

    @pl.loop(0, NPAIR)
    def _(p):
        o = base + p * PB
        gp = w * NPAIR + p
        h_is = pltpu.async_copy(idxs_hbm.at[gp], is_v, sis)
        h_ir = pltpu.async_copy(idxr_hbm.at[gp], ir_v, sir)
        h_is.wait()
        h_ga0 = pltpu.async_copy(xb_hbm.at[is_v.at[0, pl.ds(0, CB)]], ba0, ga0)
        h_ga1 = pltpu.async_copy(xb_hbm.at[is_v.at[0, pl.ds(CB, CB)]], ba1, ga1)
        h_ir.wait()
        h_gb0 = pltpu.async_copy(xc_hbm.at[ir_v.at[0, pl.ds(0, CB)]], bb0, gb0)
        h_gb1 = pltpu.async_copy(xc_hbm.at[ir_v.at[0, pl.ds(CB, CB)]], bb1, gb1)

        @pl.loop(0, PB, step=16)
        def _(j):
            idx16 = ir_v[0, pl.ds(j, 16)]
            plsc.addupdate_scatter(cnt_v, [idx16], one16)

        h_ga0.wait()
        h_wa0 = pltpu.async_copy(ba0, xbg_hbm.at[pl.ds(o, CB)], wa0)
        h_ga1.wait()
        h_wa1 = pltpu.async_copy(ba1, xbg_hbm.at[pl.ds(o + CB, CB)], wa1)
        h_gb0.wait()
        h_wb0 = pltpu.async_copy(bb0, xcg_hbm.at[pl.ds(o, CB)], wb0)
        h_gb1.wait()
        h_wb1 = pltpu.async_copy(bb1, xcg_hbm.at[pl.ds(o + CB, CB)], wb1)
        h_wa0.wait()
        h_wa1.wait()
        h_wb0.wait()
        h_wb1.wait()

    pltpu.sync_copy(cnt_v, cnt_hbm.at[pl.ds(w * NPAD, NPAD)])


def _edge_body(ea_ref, gb_ref, gc_ref, w1_ref, w2_ref, b1_ref, g_ref, b_ref,
               nea_ref, eu_ref):
    ea = ea_ref[...]
    h = jnp.dot(ea, w1_ref[...], preferred_element_type=jnp.float32)
    h = h + gb_ref[...] + gc_ref[...] + b1_ref[...]
    h = h * jax.nn.sigmoid(h)
    h = jnp.dot(h, w2_ref[...], preferred_element_type=jnp.float32)
    mu = jnp.mean(h, axis=-1, keepdims=True)
    d = h - mu
    var = jnp.mean(d * d, axis=-1, keepdims=True)
    eu = (d * lax.rsqrt(var + 1e-5)) * g_ref[...] + b_ref[...]
    eu_ref[...] = eu
    nea_ref[...] = ea + eu


def _edge(ea, gb, gc, w1a, w2, b1, g, b):
    blk = 2000
    row = lambda i: (i, 0)
    full = lambda i: (0, 0)
    return pl.pallas_call(
        _edge_body,
        grid=(N_EDGES // blk,),
        in_specs=[
            pl.BlockSpec((blk, N_HID), row),
            pl.BlockSpec((blk, N_HID), row),
            pl.BlockSpec((blk, N_HID), row),
            pl.BlockSpec((N_HID, N_HID), full),
            pl.BlockSpec((N_HID, N_HID), full),
            pl.BlockSpec((1, N_HID), full),
            pl.BlockSpec((1, N_HID), full),
            pl.BlockSpec((1, N_HID), full),
        ],
        out_specs=[
            pl.BlockSpec((blk, N_HID), row),
            pl.BlockSpec((blk, N_HID), row),
        ],
        out_shape=[jax.ShapeDtypeStruct((N_EDGES, N_HID), jnp.float32)] * 2,
    )(ea, gb, gc, w1a, w2, b1, g, b)


@functools.partial(
    pl.kernel,
    out_type=jax.ShapeDtypeStruct((NC, NPAD, N_HID), jnp.float32),
    mesh=_MESH,
    scratch_types=[
        pltpu.VMEM((CB, N_HID), jnp.float32),
        pltpu.VMEM((CB, N_HID), jnp.float32),
        pltpu.VMEM((1, CB), jnp.int32),
        pltpu.VMEM((1, CB), jnp.int32),
        pltpu.SemaphoreType.DMA,
        pltpu.SemaphoreType.DMA,
        pltpu.SemaphoreType.DMA,
        pltpu.SemaphoreType.DMA,
        pltpu.VMEM_SHARED((NPAD, N_HID), jnp.float32),
    ],
)
def _scatter_kernel(eu_hbm, idxr_hbm, sums_hbm,
                    e0, e1, i0, i1, se0, se1, si0, si1, sums_sh):
    c = lax.axis_index("c")
    s = lax.axis_index("s")
    w = c * NS + s
    zero16 = jnp.zeros((16,), jnp.float32)

    @pl.loop(0, CB)
    def _(i):
        @pl.loop(0, N_HID, step=16)
        def _(j):
            e0[i, pl.ds(j, 16)] = zero16

    @pl.loop(0, NPT, step=CB)
    def _(r):
        pltpu.sync_copy(e0, sums_sh.at[pl.ds(s * NPT + r, CB)])
    plsc.subcore_barrier()

    base = w * EPW

    @pl.loop(0, NPAIR)
    def _(p):
        o = base + p * PB
        cc2 = (w * EPW) // CB + p * 2
        h_i0 = pltpu.async_copy(idxr_hbm.at[cc2], i0, si0)
        h_e0 = pltpu.async_copy(eu_hbm.at[pl.ds(o, CB)], e0, se0)
        h_i1 = pltpu.async_copy(idxr_hbm.at[cc2 + 1], i1, si1)
        h_e1 = pltpu.async_copy(eu_hbm.at[pl.ds(o + CB, CB)], e1, se1)
        h_i0.wait()
        h_e0.wait()
        pltpu.sync_copy(e0, sums_sh.at[i0.at[0]], add=True)
        h_i1.wait()
        h_e1.wait()
        pltpu.sync_copy(e1, sums_sh.at[i1.at[0]], add=True)

    plsc.subcore_barrier()

    @pl.loop(0, NPT, step=CB)
    def _(r):
        pltpu.sync_copy(sums_sh.at[pl.ds(s * NPT + r, CB)], e0)
        pltpu.sync_copy(e0, sums_hbm.at[c, pl.ds(s * NPT + r, CB)])


def _node_body(x_ref, sums_ref, cnt_ref, w1x_ref, w1c_ref,
               b1_ref, w2_ref, g_ref, b_ref, nx_ref):
    x = x_ref[...]
    ssum = sums_ref[0] + sums_ref[1]
    cnt = jnp.maximum(jnp.sum(cnt_ref[...], axis=1)[:, None], 1.0)
    coll = ssum / cnt
    h = jnp.dot(x, w1x_ref[...], preferred_element_type=jnp.float32)
    h = h + jnp.dot(coll, w1c_ref[...], preferred_element_type=jnp.float32)
    h = h + b1_ref[...]
    h = h * jax.nn.sigmoid(h)
    h = jnp.dot(h, w2_ref[...], preferred_element_type=jnp.float32)
    mu = jnp.mean(h, axis=-1, keepdims=True)
    d = h - mu
    var = jnp.mean(d * d, axis=-1, keepdims=True)
    nx_ref[...] = x + (d * lax.rsqrt(var + 1e-5)) * g_ref[...] + b_ref[...]


def _node(x, sums, cnt, w1x, w1c, b1, w2, g, b):
    blk = 2000
    row = lambda i: (i, 0)
    full = lambda i: (0, 0)
    return pl.pallas_call(
        _node_body,
        grid=(N_NODES // blk,),
        in_specs=[
            pl.BlockSpec((blk, N_HID), row),
            pl.BlockSpec((NC, blk, N_HID), lambda i: (0, i, 0)),
            pl.BlockSpec((blk, NW), lambda i: (i, 0)),
            pl.BlockSpec((N_HID, N_HID), full),
            pl.BlockSpec((N_HID, N_HID), full),
            pl.BlockSpec((1, N_HID), full),
            pl.BlockSpec((N_HID, N_HID), full),
            pl.BlockSpec((1, N_HID), full),
            pl.BlockSpec((1, N_HID), full),
        ],
        out_specs=pl.BlockSpec((blk, N_HID), row),
        out_shape=jax.ShapeDtypeStruct((N_NODES, N_HID), jnp.float32),
    )(x, sums, cnt, w1x, w1c, b1, w2, g, b)


def kernel(edge_attr, idx_sender, idx_receiver, x_sender,
           eW1, eb1, eW2, eg, eb, rW1, rb1, rW2, rg, rb):
    ea = edge_attr.reshape(N_EDGES, N_HID)
    x = x_sender.reshape(N_NODES, N_HID)
    idx_s2 = idx_sender.astype(jnp.int32).reshape(N_EDGES // PB, 1, PB)
    idx_r2 = idx_receiver.astype(jnp.int32).reshape(N_EDGES // PB, 1, PB)
    idx_r3 = idx_receiver.astype(jnp.int32).reshape(N_EDGES // CB, 1, CB)

    w1a = eW1[:N_HID]
    xb, xc = _project(x, eW1[N_HID:2 * N_HID], eW1[2 * N_HID:])
    xbg, xcg, cnt = _gather_kernel(xb, xc, idx_s2, idx_r2)
    nea, eu = _edge(ea, xbg, xcg, w1a, eW2,
                    eb1.reshape(1, -1), eg.reshape(1, -1), eb.reshape(1, -1))
    sums = _scatter_kernel(eu, idx_r3)
    nx = _node(x, sums, cnt.reshape(NW, NPAD).T, rW1[:N_HID], rW1[N_HID:],
               rb1.reshape(1, -1), rW2, rg.reshape(1, -1), rb.reshape(1, -1))
    return (nx.reshape(1, N_NODES, N_HID), nea.reshape(1, N_EDGES, N_HID))
